# baseline (speedup 1.0000x reference)
import functools

import jax
import jax.numpy as jnp
import numpy as np
from jax import lax
from jax.experimental import pallas as pl
from jax.experimental.pallas import tpu as pltpu
from jax.experimental.pallas import tpu_sc as plsc

_NF = 26
_ED = 16
_B = 16384
_TOTAL = 26 * 100000
_D = _NF * _ED
_OFFS = np.arange(_NF, dtype=np.int32) * 100000

_ROWS = _B * _NF
_C = 104
_T = _ROWS // _C
_NW = 32
_TPW = _T // _NW
_G = 8
_NG = _TPW // _G
_GR = _G * _C


def _sc_gather(idx2d, emb_table, fc_flat):
    mesh = plsc.VectorSubcoreMesh(core_axis_name="c", subcore_axis_name="s")

    @functools.partial(
        pl.kernel,
        out_type=(
            jax.ShapeDtypeStruct((_ROWS, _ED), jnp.float32),
            jax.ShapeDtypeStruct((_ROWS,), jnp.float32),
        ),
        mesh=mesh,
        scratch_types=[
            pltpu.VMEM((_TPW, _C), jnp.int32),
            pltpu.VMEM((_GR, _ED), jnp.float32),
            pltpu.VMEM((_GR,), jnp.float32),
            pltpu.SemaphoreType.DMA,
            pltpu.SemaphoreType.DMA,
        ],
        compiler_params=pltpu.CompilerParams(use_tc_tiling_on_sc=False),
    )
    def k(idx_hbm, emb_hbm, fc_hbm, emb_out, fc_out, idx_v, ebuf, fbuf, esem, fsem):
        wid = lax.axis_index("s") * 2 + lax.axis_index("c")
        tbase = wid * _TPW
        pltpu.sync_copy(idx_hbm.at[pl.ds(tbase, _TPW)], idx_v)

        def group(g, carry):
            copies = []
            for b in range(_G):
                j = g * _G + b
                copies.append(pltpu.async_copy(
                    emb_hbm.at[idx_v.at[j]], ebuf.at[pl.ds(b * _C, _C)], esem))
                copies.append(pltpu.async_copy(
                    fc_hbm.at[idx_v.at[j]], fbuf.at[pl.ds(b * _C, _C)], fsem))
            for cp in copies:
                cp.wait()
            rbase = (tbase + g * _G) * _C
            pltpu.sync_copy(ebuf, emb_out.at[pl.ds(rbase, _GR)])
            pltpu.sync_copy(fbuf, fc_out.at[pl.ds(rbase, _GR)])
            return carry

        lax.fori_loop(0, _NG, group, 0)

    return k(idx2d, emb_table, fc_flat)


def _tc_body(e_ref, w_ref, bg_ref, fc_ref, bias_ref, out_ref):
    e = e_ref[:]
    z = jnp.dot(e, w_ref[:], preferred_element_type=jnp.float32) + bg_ref[:]
    gate = jax.nn.sigmoid(z)
    g = e * gate
    sel = (lax.broadcasted_iota(jnp.int32, (_D, _ED), 0) % _ED ==
           lax.broadcasted_iota(jnp.int32, (_D, _ED), 1)).astype(jnp.float32)
    s = jnp.dot(g, sel, preferred_element_type=jnp.float32)
    fm = 0.5 * (jnp.sum(s * s, axis=1, keepdims=True)
                - jnp.sum(g * g, axis=1, keepdims=True))
    lin = jnp.sum(fc_ref[:], axis=1, keepdims=True)
    out_ref[:] = fm + lin + bias_ref[:]


def _tc_compute(eg, fcg, W_g, b_g2, bias2, interpret=False):
    bb = 1024
    grid = _B // bb
    return pl.pallas_call(
        _tc_body,
        grid=(grid,),
        in_specs=[
            pl.BlockSpec((bb, _D), lambda i: (i, 0)),
            pl.BlockSpec((_D, _D), lambda i: (0, 0)),
            pl.BlockSpec((1, _D), lambda i: (0, 0)),
            pl.BlockSpec((bb, _NF), lambda i: (i, 0)),
            pl.BlockSpec((1, 1), lambda i: (0, 0)),
        ],
        out_specs=pl.BlockSpec((bb, 1), lambda i: (i, 0)),
        out_shape=jax.ShapeDtypeStruct((_B, 1), jnp.float32),
        interpret=interpret,
    )(eg, W_g, b_g2, fcg, bias2)


def kernel(x, emb_table, fc_table, fc_bias, W_g, b_g):
    xi = (x.astype(jnp.int32) + jnp.asarray(_OFFS)[None, :]).reshape(_T, _C)
    fc_flat = fc_table.reshape(_TOTAL)
    emb_rows, fc_rows = _sc_gather(xi, emb_table, fc_flat)
    eg = emb_rows.reshape(_B, _D)
    fcg = fc_rows.reshape(_B, _NF)
    return _tc_compute(eg, fcg, W_g, b_g.reshape(1, _D), fc_bias.reshape(1, 1))
